# Initial kernel scaffold; baseline (speedup 1.0000x reference)
import jax
import jax.numpy as jnp
from jax.experimental import pallas as pl


def kernel(x, E_absolute_position):
    raise NotImplementedError("write your pallas kernel here")


# speedup vs baseline: 1.2964x; 1.2964x over previous
import jax
import jax.numpy as jnp
from jax import lax
from jax.experimental import pallas as pl
from jax.experimental.pallas import tpu as pltpu
from jax.experimental.pallas import tpu_sc as plsc

_SEQ = 4096
_REP = 8
_D = 128
_NC = 2
_NS = 16
_NW = _NC * _NS
_ROWS = _SEQ // _REP
_TPW = _ROWS // _NW
_OPW = _SEQ // _NW
_LANES = 16


def _sc_body(table_hbm, out_hbm, in_v, out_v):
    wid = lax.axis_index("s") * _NC + lax.axis_index("c")
    pltpu.sync_copy(table_hbm.at[pl.ds(wid * _TPW, _TPW)], in_v)
    for j in range(_TPW):
        for v in range(_D // _LANES):
            vec = in_v[j, pl.ds(v * _LANES, _LANES)]
            for r in range(_REP):
                out_v[j * _REP + r, pl.ds(v * _LANES, _LANES)] = vec
    pltpu.sync_copy(out_v, out_hbm.at[pl.ds(wid * _OPW, _OPW)])


@jax.jit
def _position_encode(table):
    mesh = plsc.VectorSubcoreMesh(core_axis_name="c", subcore_axis_name="s")
    return pl.kernel(
        _sc_body,
        out_type=jax.ShapeDtypeStruct((_SEQ, _D), jnp.float32),
        mesh=mesh,
        scratch_types=[
            pltpu.VMEM((_TPW, _D), jnp.float32),
            pltpu.VMEM((_OPW, _D), jnp.float32),
        ],
    )(table)


def kernel(x, E_absolute_position):
    del x
    return _position_encode(E_absolute_position)


# speedup vs baseline: 1.4764x; 1.1388x over previous
import jax
import jax.numpy as jnp
from jax import lax
from jax.experimental import pallas as pl
from jax.experimental.pallas import tpu as pltpu
from jax.experimental.pallas import tpu_sc as plsc

_SEQ = 4096
_REP = 8
_D = 128
_NC = 2
_NS = 16
_NW = _NC * _NS
_ROWS = _SEQ // _REP
_TPW = _ROWS // _NW


def _sc_body(table_hbm, out_hbm, in_v, sem):
    wid = lax.axis_index("s") * _NC + lax.axis_index("c")
    base = wid * _TPW
    pltpu.sync_copy(table_hbm.at[pl.ds(base, _TPW)], in_v)
    copies = [
        pltpu.async_copy(in_v, out_hbm.at[pl.ds(base, _TPW), pl.ds(r, 1)], sem)
        for r in range(_REP)
    ]
    for c in copies:
        c.wait()


@jax.jit
def _position_encode(table):
    mesh = plsc.VectorSubcoreMesh(core_axis_name="c", subcore_axis_name="s")
    out = pl.kernel(
        _sc_body,
        out_type=jax.ShapeDtypeStruct((_ROWS, _REP, _D), jnp.float32),
        mesh=mesh,
        scratch_types=[
            pltpu.VMEM((_TPW, 1, _D), jnp.float32),
            pltpu.SemaphoreType.DMA,
        ],
    )(table.reshape(table.shape[0], 1, _D))
    return out.reshape(_SEQ, _D)


def kernel(x, E_absolute_position):
    del x
    return _position_encode(E_absolute_position)
